# Initial kernel scaffold; baseline (speedup 1.0000x reference)
import jax
import jax.numpy as jnp
from jax.experimental import pallas as pl


def kernel(inp, w_gate, w_noise, noise):
    raise NotImplementedError("write your pallas kernel here")


# speedup vs baseline: 5.1475x; 5.1475x over previous
import functools

import jax
import jax.numpy as jnp
from jax.experimental import pallas as pl
from jax.experimental.pallas import tpu as pltpu

D_MODEL = 1024
NUM_EXPERT = 8
TOP_K = 2
N_TOKENS = 32768
NOISE_EPS = 0.01
BLOCK_T = 2048

_INV_SQRT2 = 0.7071067811865476


def _router_body(wct_ref, inp_ref, noise_ref, idx_ref, gate_ref, loss_ref,
                 acc_imp, acc_load, *, num_blocks):
    i = pl.program_id(0)

    logits = jax.lax.dot_general(
        wct_ref[...], inp_ref[...],
        dimension_numbers=(((1,), (1,)), ((), ())),
        preferred_element_type=jnp.float32)
    clean = logits[:NUM_EXPERT, :]
    raw = logits[NUM_EXPERT:, :]
    stddev = jax.nn.softplus(raw) + NOISE_EPS
    noise_t = noise_ref[...].T
    noisy = clean + noise_t * stddev

    eidx = jax.lax.broadcasted_iota(jnp.int32, noisy.shape, 0)
    neg = jnp.float32(-jnp.inf)

    v1 = jnp.max(noisy, axis=0, keepdims=True)
    i1 = jnp.min(jnp.where(noisy == v1, eidx, NUM_EXPERT), axis=0, keepdims=True)
    m2 = jnp.where(eidx == i1, neg, noisy)
    v2 = jnp.max(m2, axis=0, keepdims=True)
    i2 = jnp.min(jnp.where(m2 == v2, eidx, NUM_EXPERT), axis=0, keepdims=True)
    m3 = jnp.where(eidx == i2, neg, m2)
    v3 = jnp.max(m3, axis=0, keepdims=True)

    a = jnp.exp(v2 - v1)
    denom = 1.0 + a
    g1 = 1.0 / denom
    g2 = a / denom

    idx_ref[...] = jnp.concatenate([i1, i2], axis=0)
    gate_ref[...] = jnp.concatenate([g1, g2], axis=0)

    thr = jnp.where(noisy > v3, v3, v2)
    z = (clean - thr) / stddev
    prob = 0.5 * (1.0 + jax.lax.erf(z * _INV_SQRT2))

    one_hot1 = jnp.where(eidx == i1, g1, 0.0)
    one_hot2 = jnp.where(eidx == i2, g2, 0.0)
    imp_part = jnp.sum(one_hot1 + one_hot2, axis=1, keepdims=True)
    load_part = jnp.sum(prob, axis=1, keepdims=True)

    @pl.when(i == 0)
    def _init():
        acc_imp[...] = jnp.zeros_like(acc_imp)
        acc_load[...] = jnp.zeros_like(acc_load)

    acc_imp[...] += imp_part
    acc_load[...] += load_part

    @pl.when(i == num_blocks - 1)
    def _fin():
        def cv_sq(x):
            mean = jnp.mean(x, keepdims=True)
            var = jnp.sum((x - mean) ** 2, keepdims=True) / (NUM_EXPERT - 1)
            return var / (mean * mean + 1e-10)
        loss_ref[...] = cv_sq(acc_imp[...]) + cv_sq(acc_load[...])


@jax.jit
def kernel(inp, w_gate, w_noise, noise):
    wct = jnp.concatenate([w_gate, w_noise], axis=1).T
    num_blocks = N_TOKENS // BLOCK_T

    idx_t, gate_t, loss = pl.pallas_call(
        functools.partial(_router_body, num_blocks=num_blocks),
        grid=(num_blocks,),
        in_specs=[
            pl.BlockSpec((2 * NUM_EXPERT, D_MODEL), lambda i: (0, 0)),
            pl.BlockSpec((BLOCK_T, D_MODEL), lambda i: (i, 0)),
            pl.BlockSpec((BLOCK_T, NUM_EXPERT), lambda i: (i, 0)),
        ],
        out_specs=[
            pl.BlockSpec((TOP_K, BLOCK_T), lambda i: (0, i)),
            pl.BlockSpec((TOP_K, BLOCK_T), lambda i: (0, i)),
            pl.BlockSpec((1, 1), lambda i: (0, 0)),
        ],
        out_shape=[
            jax.ShapeDtypeStruct((TOP_K, N_TOKENS), jnp.int32),
            jax.ShapeDtypeStruct((TOP_K, N_TOKENS), jnp.float32),
            jax.ShapeDtypeStruct((1, 1), jnp.float32),
        ],
        scratch_shapes=[
            pltpu.VMEM((NUM_EXPERT, 1), jnp.float32),
            pltpu.VMEM((NUM_EXPERT, 1), jnp.float32),
        ],
    )(wct, inp, noise)

    top_k_indices = idx_t.T.reshape(-1)
    top_k_gates = gate_t.T[:, None, :]
    return top_k_indices, top_k_gates, loss.reshape(())
